# baseline (speedup 1.0000x reference)
import functools

import jax
import jax.numpy as jnp
from jax import lax
from jax.experimental import pallas as pl
from jax.experimental.pallas import tpu as pltpu
from jax.experimental.pallas import tpu_sc as plsc

P = 2
NE = 1024
ED = 256
BETA = 0.25

M = 512
NS6 = 6


def _vq_tc_kernel(x_ref, wpe_ref, bpe_ref, g_ref, b_ref, embT_ref, emb_ref,
                  wpu_ref, bpu_ref, idx_ref, loss_ref, embp_ref):
    i = pl.program_id(0)
    n = pl.num_programs(0)
    x = x_ref[...]
    zp = jnp.dot(x, wpe_ref[...], preferred_element_type=jnp.float32) + bpe_ref[...]
    mu = jnp.mean(zp, axis=1, keepdims=True)
    zc = zp - mu
    var = jnp.mean(zc * zc, axis=1, keepdims=True)
    zp = zc / jnp.sqrt(var + 1e-5) * g_ref[...] + b_ref[...]

    emb = emb_ref[...]
    esq = jnp.sum(emb * emb, axis=1)[None, :]
    rsq = jnp.sum(zp * zp, axis=1, keepdims=True)
    scores = jnp.dot(zp, embT_ref[...], preferred_element_type=jnp.float32)
    dist = rsq + esq - 2.0 * scores
    minv = jnp.min(dist, axis=1, keepdims=True)
    cols = jax.lax.broadcasted_iota(jnp.int32, dist.shape, 1)
    idx = jnp.min(jnp.where(dist == minv, cols, NE), axis=1)
    idx_ref[0, 0, :] = idx

    s2 = jnp.sum(minv).reshape(1, 1)

    @pl.when(i == 0)
    def _():
        loss_ref[...] = s2
        embp = (jnp.dot(emb, wpu_ref[...], preferred_element_type=jnp.float32)
                + bpu_ref[...])
        for t in range(NS6):
            embp_ref[t, :, :] = embp[:, 128 * t:128 * (t + 1)]

    @pl.when(i != 0)
    def _():
        loss_ref[...] = loss_ref[...] + s2

    @pl.when(i == n - 1)
    def _():
        loss_ref[...] = loss_ref[...] * ((1.0 + BETA) / (n * M * ED))


def _make_sc_gather(B, NC, NS):
    NW = NC * NS
    bw = B // NW
    CH = 112
    nch = bw // CH
    mesh = plsc.VectorSubcoreMesh(core_axis_name="c", subcore_axis_name="s")

    @functools.partial(
        pl.kernel, mesh=mesh,
        out_type=jax.ShapeDtypeStruct((NS6, B, 128), jnp.float32),
        scratch_types=[
            pltpu.VMEM((bw,), jnp.int32),
            pltpu.VMEM((NS6, CH, 128), jnp.float32),
            pltpu.SemaphoreType.DMA,
        ],
    )
    def gather(table_hbm, idx_hbm, out_hbm, idx_v, rows_v, sem):
        wid = lax.axis_index("s") * NC + lax.axis_index("c")
        base = wid * bw
        pltpu.sync_copy(idx_hbm.at[pl.ds(base, bw)], idx_v)
        for c in range(nch):
            ids = idx_v.at[pl.ds(c * CH, CH)]
            for t in range(NS6):
                pltpu.async_copy(table_hbm.at[t].at[ids], rows_v.at[t], sem)
            for t in range(NS6):
                pltpu.make_async_copy(table_hbm.at[t].at[ids], rows_v.at[t], sem).wait()
            for t in range(NS6):
                pltpu.sync_copy(rows_v.at[t], out_hbm.at[t].at[pl.ds(base + c * CH, CH)])

    return gather


def kernel(z, emb, W_pe, b_pe, gamma, beta_ln, W_pu, b_pu):
    b, c, h, w = z.shape
    hp, wp = h // P, w // P
    D = c * P * P
    patches = z.reshape(b, c, hp, P, wp, P).transpose(0, 2, 4, 1, 3, 5).reshape(b * hp * wp, D)
    N = patches.shape[0]
    grid = N // M

    idx3, loss, embP = pl.pallas_call(
        _vq_tc_kernel,
        grid=(grid,),
        in_specs=[
            pl.BlockSpec((M, D), lambda i: (i, 0)),
            pl.BlockSpec((D, ED), lambda i: (0, 0)),
            pl.BlockSpec((1, ED), lambda i: (0, 0)),
            pl.BlockSpec((1, ED), lambda i: (0, 0)),
            pl.BlockSpec((1, ED), lambda i: (0, 0)),
            pl.BlockSpec((ED, NE), lambda i: (0, 0)),
            pl.BlockSpec((NE, ED), lambda i: (0, 0)),
            pl.BlockSpec((ED, D), lambda i: (0, 0)),
            pl.BlockSpec((1, D), lambda i: (0, 0)),
        ],
        out_specs=[
            pl.BlockSpec((1, 1, M), lambda i: (i, 0, 0)),
            pl.BlockSpec((1, 1), lambda i: (0, 0)),
            pl.BlockSpec((NS6, NE, 128), lambda i: (0, 0, 0)),
        ],
        out_shape=[
            jax.ShapeDtypeStruct((grid, 1, M), jnp.int32),
            jax.ShapeDtypeStruct((1, 1), jnp.float32),
            jax.ShapeDtypeStruct((NS6, NE, 128), jnp.float32),
        ],
    )(patches, W_pe.T, b_pe.reshape(1, ED), gamma.reshape(1, ED),
      beta_ln.reshape(1, ED), emb.T, emb, W_pu.T, b_pu.reshape(1, D))

    idx = idx3.reshape(N)
    info = plsc.get_sparse_core_info()
    out_p3 = _make_sc_gather(N, info.num_cores, info.num_subcores)(embP, idx)
    out = (out_p3.reshape(NS6, b, hp, wp, c // NS6, P, P)
           .transpose(1, 0, 4, 2, 5, 3, 6)
           .reshape(b, c, h, w))
    return out, loss[0, 0], idx


# speedup vs baseline: 1.5871x; 1.5871x over previous
import functools

import jax
import jax.numpy as jnp
from jax import lax
from jax.experimental import pallas as pl
from jax.experimental.pallas import tpu as pltpu
from jax.experimental.pallas import tpu_sc as plsc

P = 2
NE = 1024
ED = 256
BETA = 0.25

M = 448
NS6 = 6


def _vq_tc_kernel(x_ref, wpe_ref, bpe_ref, g_ref, b_ref, embT_ref, emb_ref,
                  wpu_ref, bpu_ref, idx_ref, loss_ref, embp_ref):
    pb = pl.program_id(0)
    pr = pl.program_id(1)
    first = jnp.logical_and(pb == 0, pr == 0)
    last = jnp.logical_and(pb == pl.num_programs(0) - 1, pr == pl.num_programs(1) - 1)
    ntot = pl.num_programs(0) * pl.num_programs(1)

    C = x_ref.shape[1]
    R = x_ref.shape[2] // 2
    eye = (jax.lax.broadcasted_iota(jnp.int32, (C, C), 0)
           == jax.lax.broadcasted_iota(jnp.int32, (C, C), 1)).astype(jnp.float32)
    r0 = jax.lax.broadcasted_iota(jnp.int32, (C, 4 * C), 0)
    c0 = jax.lax.broadcasted_iota(jnp.int32, (C, 4 * C), 1)
    dil = [(4 * r0 + u == c0).astype(jnp.float32) for u in range(4)]
    tiles = []
    for r in range(R):
        pieces = []
        for ki in range(2):
            row = x_ref[0, :, 2 * r + ki, :]
            rowT = jax.lax.dot_general(row, eye, (((0,), (0,)), ((), ())),
                                       preferred_element_type=jnp.float32)
            rowT3 = rowT.reshape(rowT.shape[0] // 2, 2, C)
            for kj in range(2):
                pieces.append(rowT3[:, kj, :])
        t_r = sum(jnp.dot(p, d, preferred_element_type=jnp.float32)
                  for p, d in zip(pieces, dil))
        tiles.append(t_r)
    x = jnp.concatenate(tiles, axis=0)

    zp = jnp.dot(x, wpe_ref[...], preferred_element_type=jnp.float32) + bpe_ref[...]
    mu = jnp.mean(zp, axis=1, keepdims=True)
    zc = zp - mu
    var = jnp.mean(zc * zc, axis=1, keepdims=True)
    zp = zc / jnp.sqrt(var + 1e-5) * g_ref[...] + b_ref[...]

    emb = emb_ref[...]
    esq = jnp.sum(emb * emb, axis=1)[None, :]
    rsq = jnp.sum(zp * zp, axis=1, keepdims=True)
    scores = jnp.dot(zp, embT_ref[...], preferred_element_type=jnp.float32)
    dist = rsq + esq - 2.0 * scores
    minv = jnp.min(dist, axis=1, keepdims=True)
    cols = jax.lax.broadcasted_iota(jnp.int32, dist.shape, 1)
    idx = jnp.min(jnp.where(dist == minv, cols, NE), axis=1)
    idx_ref[0, 0, :] = idx

    s2 = jnp.sum(minv).reshape(1, 1)

    @pl.when(first)
    def _():
        loss_ref[...] = s2
        embp = (jnp.dot(emb, wpu_ref[...], preferred_element_type=jnp.float32)
                + bpu_ref[...])
        for t in range(NS6):
            embp_ref[t, :, :] = embp[:, 128 * t:128 * (t + 1)]

    @pl.when(jnp.logical_not(first))
    def _():
        loss_ref[...] = loss_ref[...] + s2

    @pl.when(last)
    def _():
        loss_ref[...] = loss_ref[...] * ((1.0 + BETA) / (ntot * M * ED))


def _make_sc_gather(B, NC, NS):
    NW = NC * NS
    bw = B // NW
    CH = 112
    nch = bw // CH
    mesh = plsc.VectorSubcoreMesh(core_axis_name="c", subcore_axis_name="s")

    @functools.partial(
        pl.kernel, mesh=mesh,
        out_type=jax.ShapeDtypeStruct((NS6, B, 128), jnp.float32),
        scratch_types=[
            pltpu.VMEM((bw,), jnp.int32),
            pltpu.VMEM((NS6, CH, 128), jnp.float32),
            pltpu.SemaphoreType.DMA,
        ],
    )
    def gather(table_hbm, idx_hbm, out_hbm, idx_v, rows_v, sem):
        wid = lax.axis_index("s") * NC + lax.axis_index("c")
        base = wid * bw
        pltpu.sync_copy(idx_hbm.at[pl.ds(base, bw)], idx_v)
        for c in range(nch):
            ids = idx_v.at[pl.ds(c * CH, CH)]
            for t in range(NS6):
                pltpu.async_copy(table_hbm.at[t].at[ids], rows_v.at[t], sem)
            for t in range(NS6):
                pltpu.make_async_copy(table_hbm.at[t].at[ids], rows_v.at[t], sem).wait()
            for t in range(NS6):
                pltpu.sync_copy(rows_v.at[t], out_hbm.at[t].at[pl.ds(base + c * CH, CH)])

    return gather


def kernel(z, emb, W_pe, b_pe, gamma, beta_ln, W_pu, b_pu):
    b, c, h, w = z.shape
    hp, wp = h // P, w // P
    D = c * P * P
    N = b * hp * wp
    RB = M // wp
    nrb = hp // RB
    grid = b * nrb

    idx3, loss, embP = pl.pallas_call(
        _vq_tc_kernel,
        grid=(b, nrb),
        in_specs=[
            pl.BlockSpec((1, c, 2 * RB, w), lambda pb, pr: (pb, 0, pr, 0)),
            pl.BlockSpec((D, ED), lambda pb, pr: (0, 0)),
            pl.BlockSpec((1, ED), lambda pb, pr: (0, 0)),
            pl.BlockSpec((1, ED), lambda pb, pr: (0, 0)),
            pl.BlockSpec((1, ED), lambda pb, pr: (0, 0)),
            pl.BlockSpec((ED, NE), lambda pb, pr: (0, 0)),
            pl.BlockSpec((NE, ED), lambda pb, pr: (0, 0)),
            pl.BlockSpec((ED, D), lambda pb, pr: (0, 0)),
            pl.BlockSpec((1, D), lambda pb, pr: (0, 0)),
        ],
        out_specs=[
            pl.BlockSpec((1, 1, M), lambda pb, pr: (pb * (hp // (M // wp)) + pr, 0, 0)),
            pl.BlockSpec((1, 1), lambda pb, pr: (0, 0)),
            pl.BlockSpec((NS6, NE, 128), lambda pb, pr: (0, 0, 0)),
        ],
        out_shape=[
            jax.ShapeDtypeStruct((grid, 1, M), jnp.int32),
            jax.ShapeDtypeStruct((1, 1), jnp.float32),
            jax.ShapeDtypeStruct((NS6, NE, 128), jnp.float32),
        ],
    )(z, W_pe.T, b_pe.reshape(1, ED), gamma.reshape(1, ED),
      beta_ln.reshape(1, ED), emb.T, emb, W_pu.T, b_pu.reshape(1, D))

    idx = idx3.reshape(N)
    info = plsc.get_sparse_core_info()
    out_p3 = _make_sc_gather(N, info.num_cores, info.num_subcores)(embP, idx)
    out = (out_p3.reshape(NS6, b, hp, wp, c // NS6, P, P)
           .transpose(1, 0, 4, 2, 5, 3, 6)
           .reshape(b, c, h, w))
    return out, loss[0, 0], idx


# speedup vs baseline: 3.9641x; 2.4976x over previous
import functools

import jax
import jax.numpy as jnp
from jax import lax
from jax.experimental import pallas as pl
from jax.experimental.pallas import tpu as pltpu
from jax.experimental.pallas import tpu_sc as plsc

P = 2
NE = 1024
ED = 256
BETA = 0.25

M = 448
NS6 = 6


def _vq_tc_kernel(x_ref, wpe_ref, bpe_ref, g_ref, b_ref, embT_ref, emb_ref,
                  wpu_ref, bpu_ref, idx_ref, loss_ref, embp_ref):
    pb = pl.program_id(0)
    pr = pl.program_id(1)
    first = jnp.logical_and(pb == 0, pr == 0)
    last = jnp.logical_and(pb == pl.num_programs(0) - 1, pr == pl.num_programs(1) - 1)
    ntot = pl.num_programs(0) * pl.num_programs(1)

    C = x_ref.shape[1]
    R = x_ref.shape[2] // 2
    eye = (jax.lax.broadcasted_iota(jnp.int32, (C, C), 0)
           == jax.lax.broadcasted_iota(jnp.int32, (C, C), 1)).astype(jnp.float32)
    r0 = jax.lax.broadcasted_iota(jnp.int32, (C, 4 * C), 0)
    c0 = jax.lax.broadcasted_iota(jnp.int32, (C, 4 * C), 1)
    dil = [(4 * r0 + u == c0).astype(jnp.float32) for u in range(4)]
    tiles = []
    for r in range(R):
        pieces = []
        for ki in range(2):
            row = x_ref[0, :, 2 * r + ki, :]
            rowT = jax.lax.dot_general(row, eye, (((0,), (0,)), ((), ())),
                                       preferred_element_type=jnp.float32)
            rowT3 = rowT.reshape(rowT.shape[0] // 2, 2, C)
            for kj in range(2):
                pieces.append(rowT3[:, kj, :])
        t_r = sum(jnp.dot(p, d, preferred_element_type=jnp.float32)
                  for p, d in zip(pieces, dil))
        tiles.append(t_r)
    x = jnp.concatenate(tiles, axis=0)

    zp = jnp.dot(x, wpe_ref[...], preferred_element_type=jnp.float32) + bpe_ref[...]
    mu = jnp.mean(zp, axis=1, keepdims=True)
    zc = zp - mu
    var = jnp.mean(zc * zc, axis=1, keepdims=True)
    zp = zc / jnp.sqrt(var + 1e-5) * g_ref[...] + b_ref[...]

    emb = emb_ref[...]
    esq = jnp.sum(emb * emb, axis=1)[None, :]
    rsq = jnp.sum(zp * zp, axis=1, keepdims=True)
    scores = jnp.dot(zp, embT_ref[...], preferred_element_type=jnp.float32)
    dist = rsq + esq - 2.0 * scores
    minv = jnp.min(dist, axis=1, keepdims=True)
    cols = jax.lax.broadcasted_iota(jnp.int32, dist.shape, 1)
    idx = jnp.min(jnp.where(dist == minv, cols, NE), axis=1)
    idx_ref[0, 0, :] = idx

    s2 = jnp.sum(minv).reshape(1, 1)

    @pl.when(first)
    def _():
        loss_ref[...] = s2
        embp = (jnp.dot(emb, wpu_ref[...], preferred_element_type=jnp.float32)
                + bpu_ref[...])
        for t in range(NS6):
            embp_ref[t, :, :] = embp[:, 128 * t:128 * (t + 1)]

    @pl.when(jnp.logical_not(first))
    def _():
        loss_ref[...] = loss_ref[...] + s2

    @pl.when(last)
    def _():
        loss_ref[...] = loss_ref[...] * ((1.0 + BETA) / (ntot * M * ED))


def _unpatchify_kernel(src_ref, out_ref):
    C = out_ref.shape[1]
    W = out_ref.shape[3]
    RB = out_ref.shape[2] // 2
    wp = W // 2
    eye = (jax.lax.broadcasted_iota(jnp.int32, (wp, wp), 0)
           == jax.lax.broadcasted_iota(jnp.int32, (wp, wp), 1)).astype(jnp.float32)
    r0 = jax.lax.broadcasted_iota(jnp.int32, (wp, W), 0)
    c0 = jax.lax.broadcasted_iota(jnp.int32, (wp, W), 1)
    lace = [(2 * r0 + u == c0).astype(jnp.float32) for u in range(2)]
    for r in range(RB):
        ts = []
        for t in range(NS6):
            s = src_ref[t, wp * r:wp * (r + 1), :]
            ts.append(jax.lax.dot_general(s, eye, (((0,), (0,)), ((), ())),
                                          preferred_element_type=jnp.float32))
        T = jnp.concatenate(ts, axis=0)
        T6 = T.reshape(NS6, C // NS6, 2, 2, wp)
        for ki in range(2):
            q0 = T6[:, :, ki, 0, :].reshape(C, wp)
            q1 = T6[:, :, ki, 1, :].reshape(C, wp)
            plane = (jnp.dot(q0, lace[0], preferred_element_type=jnp.float32)
                     + jnp.dot(q1, lace[1], preferred_element_type=jnp.float32))
            out_ref[0, :, 2 * r + ki, :] = plane


def _make_sc_gather(B, NC, NS):
    NW = NC * NS
    bw = B // NW
    CH = 112
    nch = bw // CH
    mesh = plsc.VectorSubcoreMesh(core_axis_name="c", subcore_axis_name="s")

    @functools.partial(
        pl.kernel, mesh=mesh,
        out_type=jax.ShapeDtypeStruct((NS6, B, 128), jnp.float32),
        scratch_types=[
            pltpu.VMEM((bw,), jnp.int32),
            pltpu.VMEM((NS6, CH, 128), jnp.float32),
            pltpu.SemaphoreType.DMA,
        ],
    )
    def gather(table_hbm, idx_hbm, out_hbm, idx_v, rows_v, sem):
        wid = lax.axis_index("s") * NC + lax.axis_index("c")
        base = wid * bw
        pltpu.sync_copy(idx_hbm.at[pl.ds(base, bw)], idx_v)
        for c in range(nch):
            ids = idx_v.at[pl.ds(c * CH, CH)]
            for t in range(NS6):
                pltpu.async_copy(table_hbm.at[t].at[ids], rows_v.at[t], sem)
            for t in range(NS6):
                pltpu.make_async_copy(table_hbm.at[t].at[ids], rows_v.at[t], sem).wait()
            for t in range(NS6):
                pltpu.sync_copy(rows_v.at[t], out_hbm.at[t].at[pl.ds(base + c * CH, CH)])

    return gather


def kernel(z, emb, W_pe, b_pe, gamma, beta_ln, W_pu, b_pu):
    b, c, h, w = z.shape
    hp, wp = h // P, w // P
    D = c * P * P
    N = b * hp * wp
    RB = M // wp
    nrb = hp // RB
    grid = b * nrb

    idx3, loss, embP = pl.pallas_call(
        _vq_tc_kernel,
        grid=(b, nrb),
        in_specs=[
            pl.BlockSpec((1, c, 2 * RB, w), lambda pb, pr: (pb, 0, pr, 0)),
            pl.BlockSpec((D, ED), lambda pb, pr: (0, 0)),
            pl.BlockSpec((1, ED), lambda pb, pr: (0, 0)),
            pl.BlockSpec((1, ED), lambda pb, pr: (0, 0)),
            pl.BlockSpec((1, ED), lambda pb, pr: (0, 0)),
            pl.BlockSpec((ED, NE), lambda pb, pr: (0, 0)),
            pl.BlockSpec((NE, ED), lambda pb, pr: (0, 0)),
            pl.BlockSpec((ED, D), lambda pb, pr: (0, 0)),
            pl.BlockSpec((1, D), lambda pb, pr: (0, 0)),
        ],
        out_specs=[
            pl.BlockSpec((1, 1, M), lambda pb, pr: (pb * (hp // (M // wp)) + pr, 0, 0)),
            pl.BlockSpec((1, 1), lambda pb, pr: (0, 0)),
            pl.BlockSpec((NS6, NE, 128), lambda pb, pr: (0, 0, 0)),
        ],
        out_shape=[
            jax.ShapeDtypeStruct((grid, 1, M), jnp.int32),
            jax.ShapeDtypeStruct((1, 1), jnp.float32),
            jax.ShapeDtypeStruct((NS6, NE, 128), jnp.float32),
        ],
    )(z, W_pe.T, b_pe.reshape(1, ED), gamma.reshape(1, ED),
      beta_ln.reshape(1, ED), emb.T, emb, W_pu.T, b_pu.reshape(1, D))

    idx = idx3.reshape(N)
    info = plsc.get_sparse_core_info()
    out_p3 = _make_sc_gather(N, info.num_cores, info.num_subcores)(embP, idx)
    out = pl.pallas_call(
        _unpatchify_kernel,
        grid=(b, nrb),
        in_specs=[
            pl.BlockSpec((NS6, M, 128), lambda pb, pr: (0, pb * (hp // RB) + pr, 0)),
        ],
        out_specs=pl.BlockSpec((1, c, 2 * RB, w), lambda pb, pr: (pb, 0, pr, 0)),
        out_shape=jax.ShapeDtypeStruct((b, c, h, w), jnp.float32),
    )(out_p3)
    return out, loss[0, 0], idx


# speedup vs baseline: 4.2933x; 1.0830x over previous
import functools

import jax
import jax.numpy as jnp
from jax import lax
from jax.experimental import pallas as pl
from jax.experimental.pallas import tpu as pltpu
from jax.experimental.pallas import tpu_sc as plsc

P = 2
NE = 1024
ED = 256
BETA = 0.25

M = 448
NS6 = 6


def _vq_tc_kernel(x_ref, eye_ref, dil_ref, wpe_ref, bpe_ref, g_ref, b_ref,
                  embT_ref, emb_ref, wpu_ref, bpu_ref, idx_ref, loss_ref,
                  embp_ref):
    pb = pl.program_id(0)
    pr = pl.program_id(1)
    first = jnp.logical_and(pb == 0, pr == 0)
    last = jnp.logical_and(pb == pl.num_programs(0) - 1, pr == pl.num_programs(1) - 1)
    ntot = pl.num_programs(0) * pl.num_programs(1)

    C = x_ref.shape[1]
    R = x_ref.shape[2] // 2
    eye = eye_ref[...]
    dil = dil_ref[...]
    tiles = []
    for r in range(R):
        pieces = []
        for ki in range(2):
            row = x_ref[0, :, 2 * r + ki, :]
            rowT = jax.lax.dot_general(row, eye, (((0,), (0,)), ((), ())),
                                       preferred_element_type=jnp.float32)
            rowT3 = rowT.reshape(rowT.shape[0] // 2, 2, C)
            for kj in range(2):
                pieces.append(rowT3[:, kj, :])
        pcat = jnp.concatenate(pieces, axis=1)
        tiles.append(jnp.dot(pcat, dil, preferred_element_type=jnp.float32))
    x = jnp.concatenate(tiles, axis=0)

    zp = jnp.dot(x, wpe_ref[...], preferred_element_type=jnp.float32) + bpe_ref[...]
    mu = jnp.mean(zp, axis=1, keepdims=True)
    zc = zp - mu
    var = jnp.mean(zc * zc, axis=1, keepdims=True)
    zp = zc / jnp.sqrt(var + 1e-5) * g_ref[...] + b_ref[...]

    emb = emb_ref[...]
    esq = jnp.sum(emb * emb, axis=1)[None, :]
    rsq = jnp.sum(zp * zp, axis=1, keepdims=True)
    scores = jnp.dot(zp, embT_ref[...], preferred_element_type=jnp.float32)
    dist = rsq + esq - 2.0 * scores
    minv = jnp.min(dist, axis=1, keepdims=True)
    cols = jax.lax.broadcasted_iota(jnp.int32, dist.shape, 1)
    idx = jnp.min(jnp.where(dist == minv, cols, NE), axis=1)
    idx_ref[0, 0, :] = idx

    s2 = jnp.sum(minv).reshape(1, 1)

    @pl.when(first)
    def _():
        loss_ref[...] = s2
        embp = (jnp.dot(emb, wpu_ref[...], preferred_element_type=jnp.float32)
                + bpu_ref[...])
        for t in range(NS6):
            embp_ref[t, :, :] = embp[:, 128 * t:128 * (t + 1)]

    @pl.when(jnp.logical_not(first))
    def _():
        loss_ref[...] = loss_ref[...] + s2

    @pl.when(last)
    def _():
        loss_ref[...] = loss_ref[...] * ((1.0 + BETA) / (ntot * M * ED))


def _unpatchify_kernel(src_ref, eye_ref, lace_ref, out_ref):
    C = out_ref.shape[1]
    W = out_ref.shape[3]
    RB = out_ref.shape[2] // 2
    wp = W // 2
    eye = eye_ref[...]
    lace = lace_ref[...]
    for r in range(RB):
        ts = []
        for t in range(NS6):
            s = src_ref[t, wp * r:wp * (r + 1), :]
            ts.append(jax.lax.dot_general(s, eye, (((0,), (0,)), ((), ())),
                                          preferred_element_type=jnp.float32))
        T = jnp.concatenate(ts, axis=0)
        T6 = T.reshape(NS6, C // NS6, 2, 2, wp)
        for ki in range(2):
            q = jnp.concatenate([T6[:, :, ki, 0, :].reshape(C, wp),
                                 T6[:, :, ki, 1, :].reshape(C, wp)], axis=1)
            out_ref[0, :, 2 * r + ki, :] = jnp.dot(
                q, lace, preferred_element_type=jnp.float32)


def _make_sc_gather(B, NC, NS):
    NW = NC * NS
    bw = B // NW
    CH = 112
    nch = bw // CH
    mesh = plsc.VectorSubcoreMesh(core_axis_name="c", subcore_axis_name="s")

    @functools.partial(
        pl.kernel, mesh=mesh,
        out_type=jax.ShapeDtypeStruct((NS6, B, 128), jnp.float32),
        scratch_types=[
            pltpu.VMEM((bw,), jnp.int32),
            pltpu.VMEM((NS6, CH, 128), jnp.float32),
            pltpu.SemaphoreType.DMA,
        ],
    )
    def gather(table_hbm, idx_hbm, out_hbm, idx_v, rows_v, sem):
        wid = lax.axis_index("s") * NC + lax.axis_index("c")
        base = wid * bw
        pltpu.sync_copy(idx_hbm.at[pl.ds(base, bw)], idx_v)
        for c in range(nch):
            ids = idx_v.at[pl.ds(c * CH, CH)]
            for t in range(NS6):
                pltpu.async_copy(table_hbm.at[t].at[ids], rows_v.at[t], sem)
            for t in range(NS6):
                pltpu.make_async_copy(table_hbm.at[t].at[ids], rows_v.at[t], sem).wait()
            for t in range(NS6):
                pltpu.sync_copy(rows_v.at[t], out_hbm.at[t].at[pl.ds(base + c * CH, CH)])

    return gather


def kernel(z, emb, W_pe, b_pe, gamma, beta_ln, W_pu, b_pu):
    b, c, h, w = z.shape
    hp, wp = h // P, w // P
    D = c * P * P
    N = b * hp * wp
    RB = M // wp
    nrb = hp // RB
    grid = b * nrb

    eyeC = jnp.eye(c, dtype=jnp.float32)
    ar = jnp.arange(D)
    dil = ((4 * (ar % c) + ar // c)[:, None] == ar[None, :]).astype(jnp.float32)
    eyeW = jnp.eye(wp, dtype=jnp.float32)
    aw = jnp.arange(w)
    lace = ((2 * (aw % wp) + aw // wp)[:, None] == aw[None, :]).astype(jnp.float32)

    idx3, loss, embP = pl.pallas_call(
        _vq_tc_kernel,
        grid=(b, nrb),
        in_specs=[
            pl.BlockSpec((1, c, 2 * RB, w), lambda pb, pr: (pb, 0, pr, 0)),
            pl.BlockSpec((c, c), lambda pb, pr: (0, 0)),
            pl.BlockSpec((D, D), lambda pb, pr: (0, 0)),
            pl.BlockSpec((D, ED), lambda pb, pr: (0, 0)),
            pl.BlockSpec((1, ED), lambda pb, pr: (0, 0)),
            pl.BlockSpec((1, ED), lambda pb, pr: (0, 0)),
            pl.BlockSpec((1, ED), lambda pb, pr: (0, 0)),
            pl.BlockSpec((ED, NE), lambda pb, pr: (0, 0)),
            pl.BlockSpec((NE, ED), lambda pb, pr: (0, 0)),
            pl.BlockSpec((ED, D), lambda pb, pr: (0, 0)),
            pl.BlockSpec((1, D), lambda pb, pr: (0, 0)),
        ],
        out_specs=[
            pl.BlockSpec((1, 1, M), lambda pb, pr: (pb * (hp // (M // wp)) + pr, 0, 0)),
            pl.BlockSpec((1, 1), lambda pb, pr: (0, 0)),
            pl.BlockSpec((NS6, NE, 128), lambda pb, pr: (0, 0, 0)),
        ],
        out_shape=[
            jax.ShapeDtypeStruct((grid, 1, M), jnp.int32),
            jax.ShapeDtypeStruct((1, 1), jnp.float32),
            jax.ShapeDtypeStruct((NS6, NE, 128), jnp.float32),
        ],
    )(z, eyeC, dil, W_pe.T, b_pe.reshape(1, ED), gamma.reshape(1, ED),
      beta_ln.reshape(1, ED), emb.T, emb, W_pu.T, b_pu.reshape(1, D))

    idx = idx3.reshape(N)
    info = plsc.get_sparse_core_info()
    out_p3 = _make_sc_gather(N, info.num_cores, info.num_subcores)(embP, idx)
    out = pl.pallas_call(
        _unpatchify_kernel,
        grid=(b, nrb),
        in_specs=[
            pl.BlockSpec((NS6, M, 128), lambda pb, pr: (0, pb * (hp // RB) + pr, 0)),
            pl.BlockSpec((wp, wp), lambda pb, pr: (0, 0)),
            pl.BlockSpec((w, w), lambda pb, pr: (0, 0)),
        ],
        out_specs=pl.BlockSpec((1, c, 2 * RB, w), lambda pb, pr: (pb, 0, pr, 0)),
        out_shape=jax.ShapeDtypeStruct((b, c, h, w), jnp.float32),
    )(out_p3, eyeW, lace)
    return out, loss[0, 0], idx
